# baseline (speedup 1.0000x reference)
import jax
import jax.numpy as jnp
from jax.experimental import pallas as pl

D_MODEL = 1024
SEQ_BLOCK = 2048


def _add_kernel(x_ref, pos_ref, out_ref):
    out_ref[...] = x_ref[...] + pos_ref[...]


def kernel(x, pos_table):
    batch, seq_len, d_model = x.shape
    num_seq_blocks = seq_len // SEQ_BLOCK
    return pl.pallas_call(
        _add_kernel,
        grid=(num_seq_blocks, batch),
        in_specs=[
            pl.BlockSpec((1, SEQ_BLOCK, d_model), lambda i, b: (b, i, 0)),
            pl.BlockSpec((SEQ_BLOCK, d_model), lambda i, b: (i, 0)),
        ],
        out_specs=pl.BlockSpec((1, SEQ_BLOCK, d_model), lambda i, b: (b, i, 0)),
        out_shape=jax.ShapeDtypeStruct(x.shape, x.dtype),
    )(x, pos_table)
